# baseline (speedup 1.0000x reference)
import functools

import jax
import jax.numpy as jnp
from jax import lax
from jax.experimental import pallas as pl
from jax.experimental.pallas import tpu as pltpu
from jax.experimental.pallas import tpu_sc as plsc

N_NODES = 10000
HEADS = 8
OUT_CH = 16
EMBED = 128
ROWW = 144
N_EDGES = 320000

NC = 2
NS = 16
NW = NC * NS
E_PER_W = N_EDGES // NW
WIN = 80
N_FULL = E_PER_W // WIN
ROWS_A = 624
REM0 = NS * ROWS_A
REM = N_NODES - REM0

NBUF = 3
N_ROUNDS = N_FULL // NBUF
N_EPI = N_FULL - N_ROUNDS * NBUF

_f32 = jnp.float32


def _prep0_body(xf_ref, emb_ref, w_ref, m_ref, bdst_ref, h2a_ref, adt_ref):
    t = jnp.dot(emb_ref[...], w_ref[...], preferred_element_type=_f32)
    xf = xf_ref[...]
    oh = (xf == lax.broadcasted_iota(jnp.int32, (1, 8), 1)).astype(_f32)
    h2a_ref[...] = jnp.dot(oh, jnp.dot(t, m_ref[...]),
                           preferred_element_type=_f32)
    adt_ref[...] = jnp.dot(oh, jnp.dot(t, bdst_ref[...]),
                           preferred_element_type=_f32)


def _prep0(xf, emb8, W0, m0, bdst0):
    return pl.pallas_call(
        _prep0_body,
        out_shape=(
            jax.ShapeDtypeStruct((N_NODES, ROWW), _f32),
            jax.ShapeDtypeStruct((N_NODES, 16), _f32),
        ),
    )(xf, emb8, W0, m0, bdst0)


def _mid_body(p_ref, sel_ref, w_ref, m_ref, bdst_ref, h2a_ref, adt_ref):
    su = p_ref[0] + p_ref[1]
    den_rep = jnp.dot(su, sel_ref[...], preferred_element_type=_f32)
    out = su[:, :EMBED] / (den_rep + 1e-16)
    act = jnp.where(out > 0, out, jnp.exp(out) - 1.0)
    g = jnp.dot(w_ref[...], m_ref[...], preferred_element_type=_f32)
    h2a_ref[...] = jnp.dot(act, g, preferred_element_type=_f32)
    d2 = jnp.dot(w_ref[...], bdst_ref[...], preferred_element_type=_f32)
    adt_ref[...] = jnp.dot(act, d2, preferred_element_type=_f32)


def _mid(p, sel, W, m, bdst):
    return pl.pallas_call(
        _mid_body,
        out_shape=(
            jax.ShapeDtypeStruct((N_NODES, ROWW), _f32),
            jax.ShapeDtypeStruct((N_NODES, 16), _f32),
        ),
    )(p, sel, W, m, bdst)


def _final_body(p_ref, sel_ref, out_ref):
    su = p_ref[0] + p_ref[1]
    den_rep = jnp.dot(su, sel_ref[...], preferred_element_type=_f32)
    out_ref[...] = su[:, :EMBED] / (den_rep + 1e-16)


def _final(p, sel):
    return pl.pallas_call(
        _final_body,
        out_shape=jax.ShapeDtypeStruct((N_NODES, EMBED), _f32),
    )(p, sel)


def _sc_edge_pass(h2a, adt, ei, zr):
    scratch = dict(
        accum=pltpu.VMEM_SHARED((N_NODES, ROWW), _f32),
        gsem=pltpu.SemaphoreType.DMA((NBUF,)),
        ssem=pltpu.SemaphoreType.DMA((NBUF,)),
    )
    for b in range(NBUF):
        scratch['idx%d' % b] = pltpu.VMEM((2, WIN), jnp.int32)
        scratch['rows%d' % b] = pltpu.VMEM((WIN, ROWW), _f32)
        scratch['ad%d' % b] = pltpu.VMEM((WIN, 16), _f32)

    @functools.partial(
        pl.kernel,
        out_type=jax.ShapeDtypeStruct((NC, N_NODES, ROWW), _f32),
        mesh=plsc.VectorSubcoreMesh(core_axis_name="c", subcore_axis_name="s"),
        compiler_params=pltpu.CompilerParams(use_tc_tiling_on_sc=False),
        scratch_types=scratch,
    )
    def k(h2a_hbm, adt_hbm, ei_hbm, zr_hbm, out_hbm, **scr):
        c = lax.axis_index("c")
        s = lax.axis_index("s")
        wid = c * NS + s
        accum = scr['accum']
        gsem, ssem = scr['gsem'], scr['ssem']
        idx = [scr['idx%d' % b] for b in range(NBUF)]
        rows = [scr['rows%d' % b] for b in range(NBUF)]
        adb = [scr['ad%d' % b] for b in range(NBUF)]

        r0 = s * ROWS_A
        pltpu.sync_copy(zr_hbm.at[pl.ds(r0, ROWS_A)],
                        accum.at[pl.ds(r0, ROWS_A)])

        @pl.when(s == 0)
        def _():
            pltpu.sync_copy(zr_hbm.at[pl.ds(REM0, REM)],
                            accum.at[pl.ds(REM0, REM)])

        plsc.subcore_barrier()

        e0 = wid * E_PER_W

        def issue(g, b):
            base = e0 + g * WIN
            pltpu.sync_copy(ei_hbm.at[:, pl.ds(base, WIN)], idx[b])
            pltpu.async_copy(h2a_hbm.at[idx[b].at[0]], rows[b], gsem.at[b])
            pltpu.async_copy(adt_hbm.at[idx[b].at[1]], adb[b], gsem.at[b])

        def wait_gathers(b):
            pltpu.make_async_copy(h2a_hbm.at[idx[b].at[0]], rows[b],
                                  gsem.at[b]).wait()
            pltpu.make_async_copy(adt_hbm.at[idx[b].at[1]], adb[b],
                                  gsem.at[b]).wait()

        def issue_scatter(b):
            pltpu.async_copy(rows[b], accum.at[idx[b].at[1]], ssem.at[b],
                             add=True)

        def wait_scatter(b):
            pltpu.make_async_copy(rows[b], accum.at[idx[b].at[1]],
                                  ssem.at[b]).wait()

        def compute(b):
            rows_b, ad_b = rows[b], adb[b]

            @plsc.parallel_loop(0, WIN)
            def _(i):
                logit = rows_b[i, EMBED:ROWW] + ad_b[i, :]
                e = jnp.maximum(logit, 0.2 * logit)
                ex = jnp.exp(e)
                rows_b[i, EMBED:ROWW] = ex
                for h in range(HEADS):
                    rows_b[i, h * OUT_CH:(h + 1) * OUT_CH] = (
                        rows_b[i, h * OUT_CH:(h + 1) * OUT_CH] * ex[h])

        def phase(g, b):
            nb = (b + 1) % NBUF

            @pl.when(jnp.logical_and(g + 1 < N_FULL, g >= NBUF - 1))
            def _():
                wait_scatter(nb)

            @pl.when(g + 1 < N_FULL)
            def _():
                issue(g + 1, nb)

            wait_gathers(b)
            compute(b)
            issue_scatter(b)

        issue(0, 0)

        def round_body(r, _):
            g = r * NBUF
            for b in range(NBUF):
                phase(g + b, b)
            return 0

        lax.fori_loop(0, N_ROUNDS, round_body, 0)
        for j in range(N_EPI):
            phase(N_ROUNDS * NBUF + j, j)
        for j in range(NBUF):
            wait_scatter((N_FULL - NBUF + j) % NBUF)

        plsc.subcore_barrier()
        pltpu.sync_copy(accum.at[pl.ds(r0, ROWS_A)],
                        out_hbm.at[c].at[pl.ds(r0, ROWS_A)])

        @pl.when(s == 0)
        def _():
            pltpu.sync_copy(accum.at[pl.ds(REM0, REM)],
                            out_hbm.at[c].at[pl.ds(REM0, REM)])

    return k(h2a, adt, ei, zr)


def _block_diag(a, cols):
    eye = jnp.eye(HEADS, cols, dtype=_f32)
    return (a[:, :, None] * eye[:, None, :]).reshape(EMBED, cols)


def _mmat(a_src):
    return jnp.concatenate(
        [jnp.eye(EMBED, dtype=_f32), _block_diag(a_src, 16)], axis=1)


def kernel(x, edge_index, emb, W0, a_src0, a_dst0, W1, a_src1, a_dst1,
           W2, a_src2, a_dst2):
    xf = x.reshape(-1, 1).astype(jnp.int32)
    emb8 = jnp.zeros((8, EMBED), _f32).at[:5].set(emb)
    ei = edge_index.astype(jnp.int32)
    zr = jnp.zeros((N_NODES, ROWW), _f32)
    sel = jnp.concatenate(
        [jnp.zeros((EMBED, EMBED), _f32),
         jnp.kron(jnp.eye(HEADS, dtype=_f32), jnp.ones((1, OUT_CH), _f32)),
         jnp.zeros((8, EMBED), _f32)], axis=0)

    h2a, adt = _prep0(xf, emb8, W0, _mmat(a_src0), _block_diag(a_dst0, 16))
    p = _sc_edge_pass(h2a, adt, ei, zr)
    h2a, adt = _mid(p, sel, W1, _mmat(a_src1), _block_diag(a_dst1, 16))
    p = _sc_edge_pass(h2a, adt, ei, zr)
    h2a, adt = _mid(p, sel, W2, _mmat(a_src2), _block_diag(a_dst2, 16))
    p = _sc_edge_pass(h2a, adt, ei, zr)
    return _final(p, sel)


# speedup vs baseline: 1.0802x; 1.0802x over previous
import functools

import jax
import jax.numpy as jnp
from jax import lax
from jax.experimental import pallas as pl
from jax.experimental.pallas import tpu as pltpu
from jax.experimental.pallas import tpu_sc as plsc

N_NODES = 10000
HEADS = 8
OUT_CH = 16
EMBED = 128
N_EDGES = 320000

NC = 2
NS = 16
NW = NC * NS
E_PER_W = N_EDGES // NW
WIN = 80
N_FULL = E_PER_W // WIN
ROWS_A = 624
REM0 = NS * ROWS_A
REM = N_NODES - REM0

NBUF = 3
N_ROUNDS = N_FULL // NBUF
N_EPI = N_FULL - N_ROUNDS * NBUF

_f32 = jnp.float32


def _prep0_body(xf_ref, emb_ref, w_ref, bsrc_ref, bdst_ref,
                h2_ref, asrc_ref, adst_ref):
    t = jnp.dot(emb_ref[...], w_ref[...], preferred_element_type=_f32)
    xf = xf_ref[...]
    oh = (xf == lax.broadcasted_iota(jnp.int32, (1, 8), 1)).astype(_f32)
    h2 = jnp.dot(oh, t, preferred_element_type=_f32)
    h2_ref[...] = h2
    asrc_ref[...] = jnp.dot(h2, bsrc_ref[...], preferred_element_type=_f32)
    adst_ref[...] = jnp.dot(h2, bdst_ref[...], preferred_element_type=_f32)


def _prep0(xf, emb8, W0, bsrc, bdst):
    return pl.pallas_call(
        _prep0_body,
        out_shape=(
            jax.ShapeDtypeStruct((N_NODES, EMBED), _f32),
            jax.ShapeDtypeStruct((N_NODES, 16), _f32),
            jax.ShapeDtypeStruct((N_NODES, 16), _f32),
        ),
    )(xf, emb8, W0, bsrc, bdst)


def _mid_body(pm_ref, pd_ref, sel_ref, w_ref, bsrc_ref, bdst_ref,
              h2_ref, asrc_ref, adst_ref):
    u = pm_ref[0] + pm_ref[1]
    den = pd_ref[0] + pd_ref[1]
    den_rep = jnp.dot(den, sel_ref[...], preferred_element_type=_f32)
    out = u / (den_rep + 1e-16)
    act = jnp.where(out > 0, out, jnp.exp(out) - 1.0)
    h2 = jnp.dot(act, w_ref[...], preferred_element_type=_f32)
    h2_ref[...] = h2
    asrc_ref[...] = jnp.dot(h2, bsrc_ref[...], preferred_element_type=_f32)
    adst_ref[...] = jnp.dot(h2, bdst_ref[...], preferred_element_type=_f32)


def _mid(pm, pd, sel, W, bsrc, bdst):
    return pl.pallas_call(
        _mid_body,
        out_shape=(
            jax.ShapeDtypeStruct((N_NODES, EMBED), _f32),
            jax.ShapeDtypeStruct((N_NODES, 16), _f32),
            jax.ShapeDtypeStruct((N_NODES, 16), _f32),
        ),
    )(pm, pd, sel, W, bsrc, bdst)


def _final_body(pm_ref, pd_ref, sel_ref, out_ref):
    u = pm_ref[0] + pm_ref[1]
    den = pd_ref[0] + pd_ref[1]
    den_rep = jnp.dot(den, sel_ref[...], preferred_element_type=_f32)
    out_ref[...] = u / (den_rep + 1e-16)


def _final(pm, pd, sel):
    return pl.pallas_call(
        _final_body,
        out_shape=jax.ShapeDtypeStruct((N_NODES, EMBED), _f32),
    )(pm, pd, sel)


def _sc_edge_pass(h2t, asrct, adstt, ei, zm, zd):
    scratch = dict(
        accum_m=pltpu.VMEM_SHARED((N_NODES, EMBED), _f32),
        accum_d=pltpu.VMEM_SHARED((N_NODES, 16), _f32),
        gsem=pltpu.SemaphoreType.DMA((NBUF,)),
        ssem=pltpu.SemaphoreType.DMA((NBUF,)),
    )
    for b in range(NBUF):
        scratch['idx%d' % b] = pltpu.VMEM((2, WIN), jnp.int32)
        scratch['rows%d' % b] = pltpu.VMEM((WIN, EMBED), _f32)
        scratch['as%d' % b] = pltpu.VMEM((WIN, 16), _f32)
        scratch['ad%d' % b] = pltpu.VMEM((WIN, 16), _f32)

    @functools.partial(
        pl.kernel,
        out_type=(
            jax.ShapeDtypeStruct((NC, N_NODES, EMBED), _f32),
            jax.ShapeDtypeStruct((NC, N_NODES, 16), _f32),
        ),
        mesh=plsc.VectorSubcoreMesh(core_axis_name="c", subcore_axis_name="s"),
        compiler_params=pltpu.CompilerParams(use_tc_tiling_on_sc=False),
        scratch_types=scratch,
    )
    def k(h2_hbm, asrc_hbm, adst_hbm, ei_hbm, zm_hbm, zd_hbm,
          outm_hbm, outd_hbm, **scr):
        c = lax.axis_index("c")
        s = lax.axis_index("s")
        wid = c * NS + s
        accum_m, accum_d = scr['accum_m'], scr['accum_d']
        gsem, ssem = scr['gsem'], scr['ssem']
        idx = [scr['idx%d' % b] for b in range(NBUF)]
        rows = [scr['rows%d' % b] for b in range(NBUF)]
        asb = [scr['as%d' % b] for b in range(NBUF)]
        adb = [scr['ad%d' % b] for b in range(NBUF)]

        r0 = s * ROWS_A
        pltpu.sync_copy(zm_hbm.at[pl.ds(r0, ROWS_A)],
                        accum_m.at[pl.ds(r0, ROWS_A)])
        pltpu.sync_copy(zd_hbm.at[pl.ds(r0, ROWS_A)],
                        accum_d.at[pl.ds(r0, ROWS_A)])

        @pl.when(s == 0)
        def _():
            pltpu.sync_copy(zm_hbm.at[pl.ds(REM0, REM)],
                            accum_m.at[pl.ds(REM0, REM)])
            pltpu.sync_copy(zd_hbm.at[pl.ds(REM0, REM)],
                            accum_d.at[pl.ds(REM0, REM)])

        plsc.subcore_barrier()

        e0 = wid * E_PER_W

        def issue(g, b):
            base = e0 + g * WIN
            pltpu.sync_copy(ei_hbm.at[:, pl.ds(base, WIN)], idx[b])
            pltpu.async_copy(h2_hbm.at[idx[b].at[0]], rows[b], gsem.at[b])
            pltpu.async_copy(asrc_hbm.at[idx[b].at[0]], asb[b], gsem.at[b])
            pltpu.async_copy(adst_hbm.at[idx[b].at[1]], adb[b], gsem.at[b])

        def wait_gathers(b):
            pltpu.make_async_copy(h2_hbm.at[idx[b].at[0]], rows[b],
                                  gsem.at[b]).wait()
            pltpu.make_async_copy(asrc_hbm.at[idx[b].at[0]], asb[b],
                                  gsem.at[b]).wait()
            pltpu.make_async_copy(adst_hbm.at[idx[b].at[1]], adb[b],
                                  gsem.at[b]).wait()

        def issue_scatter(b):
            pltpu.async_copy(rows[b], accum_m.at[idx[b].at[1]], ssem.at[b],
                             add=True)
            pltpu.async_copy(asb[b], accum_d.at[idx[b].at[1]], ssem.at[b],
                             add=True)

        def wait_scatter(b):
            pltpu.make_async_copy(rows[b], accum_m.at[idx[b].at[1]],
                                  ssem.at[b]).wait()
            pltpu.make_async_copy(asb[b], accum_d.at[idx[b].at[1]],
                                  ssem.at[b]).wait()

        def compute(b):
            rows_b, as_b, ad_b = rows[b], asb[b], adb[b]

            @plsc.parallel_loop(0, WIN, unroll=2)
            def _(i):
                logit = as_b[i, :] + ad_b[i, :]
                e = jnp.maximum(logit, 0.2 * logit)
                ex = jnp.exp(e)
                as_b[i, :] = ex
                for h in range(HEADS):
                    rows_b[i, h * OUT_CH:(h + 1) * OUT_CH] = (
                        rows_b[i, h * OUT_CH:(h + 1) * OUT_CH] * ex[h])

        def phase(g, b):
            nb = (b + 1) % NBUF

            @pl.when(jnp.logical_and(g + 1 < N_FULL, g >= NBUF - 1))
            def _():
                wait_scatter(nb)

            @pl.when(g + 1 < N_FULL)
            def _():
                issue(g + 1, nb)

            wait_gathers(b)
            compute(b)
            issue_scatter(b)

        issue(0, 0)

        def round_body(r, _):
            g = r * NBUF
            for b in range(NBUF):
                phase(g + b, b)
            return 0

        lax.fori_loop(0, N_ROUNDS, round_body, 0)
        for j in range(N_EPI):
            phase(N_ROUNDS * NBUF + j, j)
        for j in range(NBUF):
            wait_scatter((N_FULL - NBUF + j) % NBUF)

        plsc.subcore_barrier()
        pltpu.sync_copy(accum_m.at[pl.ds(r0, ROWS_A)],
                        outm_hbm.at[c].at[pl.ds(r0, ROWS_A)])
        pltpu.sync_copy(accum_d.at[pl.ds(r0, ROWS_A)],
                        outd_hbm.at[c].at[pl.ds(r0, ROWS_A)])

        @pl.when(s == 0)
        def _():
            pltpu.sync_copy(accum_m.at[pl.ds(REM0, REM)],
                            outm_hbm.at[c].at[pl.ds(REM0, REM)])
            pltpu.sync_copy(accum_d.at[pl.ds(REM0, REM)],
                            outd_hbm.at[c].at[pl.ds(REM0, REM)])

    return k(h2t, asrct, adstt, ei, zm, zd)


def _block_diag(a):
    eye = jnp.eye(HEADS, 16, dtype=_f32)
    return (a[:, :, None] * eye[:, None, :]).reshape(EMBED, 16)


def kernel(x, edge_index, emb, W0, a_src0, a_dst0, W1, a_src1, a_dst1,
           W2, a_src2, a_dst2):
    xf = x.reshape(-1, 1).astype(jnp.int32)
    emb8 = jnp.zeros((8, EMBED), _f32).at[:5].set(emb)
    ei = edge_index.astype(jnp.int32)
    zm = jnp.zeros((N_NODES, EMBED), _f32)
    zd = jnp.zeros((N_NODES, 16), _f32)
    sel = jnp.concatenate(
        [jnp.kron(jnp.eye(HEADS, dtype=_f32), jnp.ones((1, OUT_CH), _f32)),
         jnp.zeros((8, EMBED), _f32)], axis=0)

    h2t, asrct, adstt = _prep0(xf, emb8, W0, _block_diag(a_src0),
                               _block_diag(a_dst0))
    pm, pd = _sc_edge_pass(h2t, asrct, adstt, ei, zm, zd)
    h2t, asrct, adstt = _mid(pm, pd, sel, W1, _block_diag(a_src1),
                             _block_diag(a_dst1))
    pm, pd = _sc_edge_pass(h2t, asrct, adstt, ei, zm, zd)
    h2t, asrct, adstt = _mid(pm, pd, sel, W2, _block_diag(a_src2),
                             _block_diag(a_dst2))
    pm, pd = _sc_edge_pass(h2t, asrct, adstt, ei, zm, zd)
    return _final(pm, pd, sel)


# speedup vs baseline: 1.2037x; 1.1144x over previous
import functools

import jax
import jax.numpy as jnp
from jax import lax
from jax.experimental import pallas as pl
from jax.experimental.pallas import tpu as pltpu
from jax.experimental.pallas import tpu_sc as plsc

N_NODES = 10000
HEADS = 8
OUT_CH = 16
EMBED = 128
N_EDGES = 320000

NC = 2
NS = 16
NW = NC * NS
E_PER_W = N_EDGES // NW
WIN = 80
N_FULL = E_PER_W // WIN
ROWS_A = 624
REM0 = NS * ROWS_A
REM = N_NODES - REM0

NBUF = 3
N_ROUNDS = N_FULL // NBUF
N_EPI = N_FULL - N_ROUNDS * NBUF

_f32 = jnp.float32


def _prep0_body(xf_ref, emb_ref, w_ref, afs_ref, afd_ref, k_ref,
                h2_ref, asrc_ref, adst_ref):
    t = jnp.dot(emb_ref[...], w_ref[...], preferred_element_type=_f32)
    xf = xf_ref[...]
    oh = (xf == lax.broadcasted_iota(jnp.int32, (1, 8), 1)).astype(_f32)
    h2 = jnp.dot(oh, t, preferred_element_type=_f32)
    h2_ref[...] = h2
    k = k_ref[...]
    asrc_ref[...] = jnp.dot(h2 * afs_ref[...], k, preferred_element_type=_f32)
    adst_ref[...] = jnp.dot(h2 * afd_ref[...], k, preferred_element_type=_f32)


def _prep0(xf, emb8, W0, afs, afd, kmat):
    return pl.pallas_call(
        _prep0_body,
        out_shape=(
            jax.ShapeDtypeStruct((N_NODES, EMBED), _f32),
            jax.ShapeDtypeStruct((N_NODES, 16), _f32),
            jax.ShapeDtypeStruct((N_NODES, 16), _f32),
        ),
    )(xf, emb8, W0, afs, afd, kmat)


def _mid_body(pm_ref, pd_ref, sel_ref, w_ref, afs_ref, afd_ref, k_ref,
              h2_ref, asrc_ref, adst_ref):
    u = pm_ref[0] + pm_ref[1]
    den = pd_ref[0] + pd_ref[1]
    den_rep = jnp.dot(den, sel_ref[...], preferred_element_type=_f32)
    out = u / (den_rep + 1e-16)
    act = jnp.where(out > 0, out, jnp.exp(out) - 1.0)
    h2 = jnp.dot(act, w_ref[...], preferred_element_type=_f32)
    h2_ref[...] = h2
    k = k_ref[...]
    asrc_ref[...] = jnp.dot(h2 * afs_ref[...], k, preferred_element_type=_f32)
    adst_ref[...] = jnp.dot(h2 * afd_ref[...], k, preferred_element_type=_f32)


def _mid(pm, pd, sel, W, afs, afd, kmat):
    return pl.pallas_call(
        _mid_body,
        out_shape=(
            jax.ShapeDtypeStruct((N_NODES, EMBED), _f32),
            jax.ShapeDtypeStruct((N_NODES, 16), _f32),
            jax.ShapeDtypeStruct((N_NODES, 16), _f32),
        ),
    )(pm, pd, sel, W, afs, afd, kmat)


def _final_body(pm_ref, pd_ref, sel_ref, out_ref):
    u = pm_ref[0] + pm_ref[1]
    den = pd_ref[0] + pd_ref[1]
    den_rep = jnp.dot(den, sel_ref[...], preferred_element_type=_f32)
    out_ref[...] = u / (den_rep + 1e-16)


def _final(pm, pd, sel):
    return pl.pallas_call(
        _final_body,
        out_shape=jax.ShapeDtypeStruct((N_NODES, EMBED), _f32),
    )(pm, pd, sel)


def _sc_edge_pass(h2t, asrct, adstt, ei, zm, zd):
    scratch = dict(
        accum_m=pltpu.VMEM_SHARED((N_NODES, EMBED), _f32),
        accum_d=pltpu.VMEM_SHARED((N_NODES, 16), _f32),
        gsem=pltpu.SemaphoreType.DMA((NBUF,)),
        ssem=pltpu.SemaphoreType.DMA((NBUF,)),
        isem=pltpu.SemaphoreType.DMA((NBUF,)),
    )
    for b in range(NBUF):
        scratch['idx%d' % b] = pltpu.VMEM((2, WIN), jnp.int32)
        scratch['sidx%d' % b] = pltpu.VMEM((WIN,), jnp.int32)
        scratch['rows%d' % b] = pltpu.VMEM((WIN, EMBED), _f32)
        scratch['as%d' % b] = pltpu.VMEM((WIN, 16), _f32)
        scratch['ad%d' % b] = pltpu.VMEM((WIN, 16), _f32)

    @functools.partial(
        pl.kernel,
        out_type=(
            jax.ShapeDtypeStruct((NC, N_NODES, EMBED), _f32),
            jax.ShapeDtypeStruct((NC, N_NODES, 16), _f32),
        ),
        mesh=plsc.VectorSubcoreMesh(core_axis_name="c", subcore_axis_name="s"),
        compiler_params=pltpu.CompilerParams(use_tc_tiling_on_sc=False),
        scratch_types=scratch,
    )
    def k(h2_hbm, asrc_hbm, adst_hbm, ei_hbm, zm_hbm, zd_hbm,
          outm_hbm, outd_hbm, **scr):
        c = lax.axis_index("c")
        s = lax.axis_index("s")
        wid = c * NS + s
        accum_m, accum_d = scr['accum_m'], scr['accum_d']
        gsem, ssem, isem = scr['gsem'], scr['ssem'], scr['isem']
        idx = [scr['idx%d' % b] for b in range(NBUF)]
        sidx = [scr['sidx%d' % b] for b in range(NBUF)]
        rows = [scr['rows%d' % b] for b in range(NBUF)]
        asb = [scr['as%d' % b] for b in range(NBUF)]
        adb = [scr['ad%d' % b] for b in range(NBUF)]

        r0 = s * ROWS_A
        pltpu.sync_copy(zm_hbm.at[pl.ds(r0, ROWS_A)],
                        accum_m.at[pl.ds(r0, ROWS_A)])
        pltpu.sync_copy(zd_hbm.at[pl.ds(r0, ROWS_A)],
                        accum_d.at[pl.ds(r0, ROWS_A)])

        @pl.when(s == 0)
        def _():
            pltpu.sync_copy(zm_hbm.at[pl.ds(REM0, REM)],
                            accum_m.at[pl.ds(REM0, REM)])
            pltpu.sync_copy(zd_hbm.at[pl.ds(REM0, REM)],
                            accum_d.at[pl.ds(REM0, REM)])

        plsc.subcore_barrier()

        e0 = wid * E_PER_W

        def issue_idx(g, b):
            base = e0 + g * WIN
            pltpu.async_copy(ei_hbm.at[:, pl.ds(base, WIN)], idx[b],
                             isem.at[b])

        def wait_idx(g, b):
            base = e0 + g * WIN
            pltpu.make_async_copy(ei_hbm.at[:, pl.ds(base, WIN)], idx[b],
                                  isem.at[b]).wait()

        def issue_gathers(b):
            pltpu.async_copy(h2_hbm.at[idx[b].at[0]], rows[b], gsem.at[b])
            pltpu.async_copy(asrc_hbm.at[idx[b].at[0]], asb[b], gsem.at[b])
            pltpu.async_copy(adst_hbm.at[idx[b].at[1]], adb[b], gsem.at[b])

        def wait_gathers(b):
            pltpu.make_async_copy(h2_hbm.at[idx[b].at[0]], rows[b],
                                  gsem.at[b]).wait()
            pltpu.make_async_copy(asrc_hbm.at[idx[b].at[0]], asb[b],
                                  gsem.at[b]).wait()
            pltpu.make_async_copy(adst_hbm.at[idx[b].at[1]], adb[b],
                                  gsem.at[b]).wait()

        def issue_scatter(b):
            pltpu.async_copy(rows[b], accum_m.at[sidx[b]], ssem.at[b],
                             add=True)
            pltpu.async_copy(asb[b], accum_d.at[sidx[b]], ssem.at[b],
                             add=True)

        def wait_scatter(b):
            pltpu.make_async_copy(rows[b], accum_m.at[sidx[b]],
                                  ssem.at[b]).wait()
            pltpu.make_async_copy(asb[b], accum_d.at[sidx[b]],
                                  ssem.at[b]).wait()

        def compute(b):
            rows_b, as_b, ad_b = rows[b], asb[b], adb[b]
            idx_b, sidx_b = idx[b], sidx[b]

            for t in range(WIN // 16):
                sidx_b[t * 16:(t + 1) * 16] = idx_b[1, t * 16:(t + 1) * 16]

            @plsc.parallel_loop(0, WIN, unroll=2)
            def _(i):
                logit = as_b[i, :] + ad_b[i, :]
                e = jnp.maximum(logit, 0.2 * logit)
                ex = jnp.exp(e)
                as_b[i, :] = ex
                for h in range(HEADS):
                    rows_b[i, h * OUT_CH:(h + 1) * OUT_CH] = (
                        rows_b[i, h * OUT_CH:(h + 1) * OUT_CH] * ex[h])

        def phase(g, b):
            nb = (b + 1) % NBUF
            pb = (b + 2) % NBUF

            @pl.when(g + 2 < N_FULL)
            def _():
                issue_idx(g + 2, pb)

            @pl.when(jnp.logical_and(g + 1 < N_FULL, g >= NBUF - 1))
            def _():
                wait_scatter(nb)

            @pl.when(g + 1 < N_FULL)
            def _():
                wait_idx(g + 1, nb)
                issue_gathers(nb)

            wait_gathers(b)
            compute(b)
            issue_scatter(b)

        issue_idx(0, 0)
        issue_idx(1, 1)
        wait_idx(0, 0)
        issue_gathers(0)

        def round_body(r, _):
            g = r * NBUF
            for b in range(NBUF):
                phase(g + b, b)
            return 0

        lax.fori_loop(0, N_ROUNDS, round_body, 0)
        for j in range(N_EPI):
            phase(N_ROUNDS * NBUF + j, j)
        for j in range(NBUF):
            wait_scatter((N_FULL - NBUF + j) % NBUF)

        plsc.subcore_barrier()
        pltpu.sync_copy(accum_m.at[pl.ds(r0, ROWS_A)],
                        outm_hbm.at[c].at[pl.ds(r0, ROWS_A)])
        pltpu.sync_copy(accum_d.at[pl.ds(r0, ROWS_A)],
                        outd_hbm.at[c].at[pl.ds(r0, ROWS_A)])

        @pl.when(s == 0)
        def _():
            pltpu.sync_copy(accum_m.at[pl.ds(REM0, REM)],
                            outm_hbm.at[c].at[pl.ds(REM0, REM)])
            pltpu.sync_copy(accum_d.at[pl.ds(REM0, REM)],
                            outd_hbm.at[c].at[pl.ds(REM0, REM)])

    return k(h2t, asrct, adstt, ei, zm, zd)


def kernel(x, edge_index, emb, W0, a_src0, a_dst0, W1, a_src1, a_dst1,
           W2, a_src2, a_dst2):
    xf = x.reshape(-1, 1).astype(jnp.int32)
    emb8 = jnp.zeros((8, EMBED), _f32).at[:5].set(emb)
    ei = edge_index.astype(jnp.int32)
    zm = jnp.zeros((N_NODES, EMBED), _f32)
    zd = jnp.zeros((N_NODES, 16), _f32)
    sel = jnp.concatenate(
        [jnp.kron(jnp.eye(HEADS, dtype=_f32), jnp.ones((1, OUT_CH), _f32)),
         jnp.zeros((8, EMBED), _f32)], axis=0)
    kmat = jnp.kron(jnp.eye(HEADS, 16, dtype=_f32),
                    jnp.ones((OUT_CH, 1), _f32))
    afs = [a_src0.reshape(1, EMBED), a_src1.reshape(1, EMBED),
           a_src2.reshape(1, EMBED)]
    afd = [a_dst0.reshape(1, EMBED), a_dst1.reshape(1, EMBED),
           a_dst2.reshape(1, EMBED)]

    h2t, asrct, adstt = _prep0(xf, emb8, W0, afs[0], afd[0], kmat)
    pm, pd = _sc_edge_pass(h2t, asrct, adstt, ei, zm, zd)
    h2t, asrct, adstt = _mid(pm, pd, sel, W1, afs[1], afd[1], kmat)
    pm, pd = _sc_edge_pass(h2t, asrct, adstt, ei, zm, zd)
    h2t, asrct, adstt = _mid(pm, pd, sel, W2, afs[2], afd[2], kmat)
    pm, pd = _sc_edge_pass(h2t, asrct, adstt, ei, zm, zd)
    return _final(pm, pd, sel)
